# Initial kernel scaffold; baseline (speedup 1.0000x reference)
import jax
import jax.numpy as jnp
from jax.experimental import pallas as pl


def kernel(X, edge_index, edge_weight, Wxz, bxz, Whz, bhz, Wxr, bxr, Whr, bhr, Wxh, bxh, Whh, bhh):
    raise NotImplementedError("write your pallas kernel here")


# speedup vs baseline: 1.1285x; 1.1285x over previous
import functools

import jax
import jax.numpy as jnp
from jax.experimental import pallas as pl

N_NODES = 10000
IN_CH = 128
HID = 128
ROW_BLK = 1000


def _scatter_batch(v, src, dst, w):
    return jax.ops.segment_sum(w[:, None] * v[src], dst, num_segments=N_NODES)


def _gru_zr_body(gxzr_ref, hmm_ref, h_ref, z_ref, hr_ref):
    zr = jax.nn.sigmoid(gxzr_ref[...] + hmm_ref[...])
    z = zr[:, :HID]
    r = zr[:, HID:]
    z_ref[...] = z
    hr_ref[...] = h_ref[...] * r


def _gru_h_body(gxh_ref, hrmm_ref, z_ref, h_ref, out_ref):
    htil = jnp.tanh(gxh_ref[...] + hrmm_ref[...])
    z = z_ref[...]
    out_ref[...] = z * h_ref[...] + (1.0 - z) * htil


def _row_spec(c):
    return pl.BlockSpec((ROW_BLK, c), lambda i: (i, 0))


_gru_zr = pl.pallas_call(
    _gru_zr_body,
    grid=(N_NODES // ROW_BLK,),
    in_specs=[_row_spec(2 * HID), _row_spec(2 * HID), _row_spec(HID)],
    out_specs=[_row_spec(HID), _row_spec(HID)],
    out_shape=[
        jax.ShapeDtypeStruct((N_NODES, HID), jnp.float32),
        jax.ShapeDtypeStruct((N_NODES, HID), jnp.float32),
    ],
)

_gru_h = pl.pallas_call(
    _gru_h_body,
    grid=(N_NODES // ROW_BLK,),
    in_specs=[_row_spec(HID), _row_spec(HID), _row_spec(HID), _row_spec(HID)],
    out_specs=_row_spec(HID),
    out_shape=jax.ShapeDtypeStruct((N_NODES, HID), jnp.float32),
)


def _cat_weights(W, b):
    return jnp.concatenate([W[0] - W[2], W[1], 2.0 * W[2]], axis=0), b


def kernel(X, edge_index, edge_weight, Wxz, bxz, Whz, bhz, Wxr, bxr, Whr, bhr,
           Wxh, bxh, Whh, bhh):
    src = edge_index[0]
    dst = edge_index[1]
    deg = jax.ops.segment_sum(edge_weight, src, num_segments=N_NODES)
    dinv = jnp.where(deg > 0, deg ** -0.5, 0.0)
    w = -dinv[src] * edge_weight * dinv[dst]

    batches, seq_len, num_nodes, _ = X.shape
    S = functools.partial(_scatter_batch, src=src, dst=dst, w=w)

    X0 = jnp.transpose(X[0], (1, 0, 2)).reshape(num_nodes, seq_len * IN_CH)
    X1 = S(X0)
    X2 = S(X1)
    Wx_cat = jnp.concatenate([_cat_weights(Wxz, bxz)[0],
                              _cat_weights(Wxr, bxr)[0],
                              _cat_weights(Wxh, bxh)[0]], axis=1)
    bx_cat = jnp.concatenate([bxz, bxr, bxh])
    Whzr_cat = jnp.concatenate([_cat_weights(Whz, bhz)[0],
                                _cat_weights(Whr, bhr)[0]], axis=1)
    bh_zr = jnp.concatenate([bhz, bhr])
    Whh_cat = _cat_weights(Whh, bhh)[0]

    def xgate(t):
        basis = jnp.concatenate(
            [X0[:, t * IN_CH:(t + 1) * IN_CH],
             X1[:, t * IN_CH:(t + 1) * IN_CH],
             X2[:, t * IN_CH:(t + 1) * IN_CH]], axis=1)
        return basis @ Wx_cat + bx_cat

    Gx = [xgate(t) for t in range(seq_len)]

    H = jnp.zeros((num_nodes, HID), dtype=X.dtype)
    states = []
    for t in range(seq_len):
        H1 = S(H)
        H2 = S(H1)
        Bh = jnp.concatenate([H, H1, H2], axis=1)
        hmm = Bh @ Whzr_cat + bh_zr
        Z, HR = _gru_zr(Gx[t][:, :2 * HID], hmm, H)
        R1 = S(HR)
        R2 = S(R1)
        Bhr = jnp.concatenate([HR, R1, R2], axis=1)
        hrmm = Bhr @ Whh_cat + bhh
        H = _gru_h(Gx[t][:, 2 * HID:], hrmm, Z, H)
        states.append(H)

    out = jnp.stack(states, axis=0)[None]
    return out, H[None]


# speedup vs baseline: 3.3701x; 2.9864x over previous
import functools

import jax
import jax.numpy as jnp
from jax import lax
from jax.experimental import pallas as pl
from jax.experimental.pallas import tpu as pltpu
from jax.experimental.pallas import tpu_sc as plsc

N_NODES = 10000
IN_CH = 128
HID = 128
ROW_BLK = 1000

N_EDGES = 320000
NC = 2
NS = 16
NW = NC * NS
EPW = N_EDGES // NW
CHUNK = 80
NCHUNK = EPW // CHUNK
NSEG = 5
SEGCHUNK = NCHUNK // NSEG
ZROWS = 80
NBLK = N_NODES // ZROWS

_sc_mesh = plsc.VectorSubcoreMesh(core_axis_name="c", subcore_axis_name="s")


@functools.partial(
    pl.kernel,
    out_type=jax.ShapeDtypeStruct((NC, N_NODES, HID), jnp.float32),
    mesh=_sc_mesh,
    scratch_types=[
        pltpu.VMEM((SEGCHUNK, CHUNK), jnp.int32),
        pltpu.VMEM((SEGCHUNK, CHUNK), jnp.int32),
        pltpu.VMEM((SEGCHUNK, CHUNK), jnp.float32),
        pltpu.VMEM((CHUNK, HID), jnp.float32),
        pltpu.VMEM_SHARED((N_NODES, HID), jnp.float32),
        pltpu.SemaphoreType.DMA,
    ],
)
def _lap_sc(v_hbm, src_hbm, dst_hbm, w_hbm, out_hbm,
            src_t, dst_t, w_t, rows, acc, sem):
    cid = lax.axis_index("c")
    sid = lax.axis_index("s")
    wid = sid * NC + cid

    zeros16 = jnp.zeros((16,), jnp.float32)

    def zrow(i, _):
        for k in range(HID // 16):
            rows[i, pl.ds(k * 16, 16)] = zeros16
        return 0

    lax.fori_loop(0, ZROWS, zrow, 0)
    for j in range((NBLK + NS - 1) // NS):
        b = sid + j * NS

        @pl.when(b < NBLK)
        def _():
            off = pl.multiple_of(b * ZROWS, ZROWS)
            pltpu.sync_copy(rows, acc.at[pl.ds(off, ZROWS)])

    plsc.subcore_barrier()

    for seg in range(NSEG):
        pltpu.sync_copy(src_hbm.at[wid, seg], src_t)
        pltpu.sync_copy(dst_hbm.at[wid, seg], dst_t)
        pltpu.sync_copy(w_hbm.at[wid, seg], w_t)

        def chunk_body(c, _):
            pltpu.async_copy(v_hbm.at[src_t.at[c]], rows, sem).wait()

            def group_body(g, _):
                wv = w_t[c, pl.ds(g * 16, 16)]
                for j in range(16):
                    we = wv[j]
                    e = g * 16 + j
                    for k in range(HID // 16):
                        sl = pl.ds(k * 16, 16)
                        rows[e, sl] = rows[e, sl] * we
                return 0

            lax.fori_loop(0, CHUNK // 16, group_body, 0)
            pltpu.sync_copy(rows, acc.at[dst_t.at[c]], add=True)
            return 0

        lax.fori_loop(0, SEGCHUNK, chunk_body, 0)

    plsc.subcore_barrier()

    for j in range((NBLK + NS - 1) // NS):
        b = sid + j * NS

        @pl.when(b < NBLK)
        def _():
            off = pl.multiple_of(b * ZROWS, ZROWS)
            pltpu.sync_copy(acc.at[pl.ds(off, ZROWS)],
                            out_hbm.at[cid, pl.ds(off, ZROWS)])


def _gru_zr_body(gxzr_ref, hmm_ref, h_ref, z_ref, hr_ref):
    zr = jax.nn.sigmoid(gxzr_ref[...] + hmm_ref[...])
    z = zr[:, :HID]
    r = zr[:, HID:]
    z_ref[...] = z
    hr_ref[...] = h_ref[...] * r


def _gru_h_body(gxh_ref, hrmm_ref, z_ref, h_ref, out_ref):
    htil = jnp.tanh(gxh_ref[...] + hrmm_ref[...])
    z = z_ref[...]
    out_ref[...] = z * h_ref[...] + (1.0 - z) * htil


def _row_spec(c):
    return pl.BlockSpec((ROW_BLK, c), lambda i: (i, 0))


_gru_zr = pl.pallas_call(
    _gru_zr_body,
    grid=(N_NODES // ROW_BLK,),
    in_specs=[_row_spec(2 * HID), _row_spec(2 * HID), _row_spec(HID)],
    out_specs=[_row_spec(HID), _row_spec(HID)],
    out_shape=[
        jax.ShapeDtypeStruct((N_NODES, HID), jnp.float32),
        jax.ShapeDtypeStruct((N_NODES, HID), jnp.float32),
    ],
)

_gru_h = pl.pallas_call(
    _gru_h_body,
    grid=(N_NODES // ROW_BLK,),
    in_specs=[_row_spec(HID), _row_spec(HID), _row_spec(HID), _row_spec(HID)],
    out_specs=_row_spec(HID),
    out_shape=jax.ShapeDtypeStruct((N_NODES, HID), jnp.float32),
)


def _cat_weights(W):
    return jnp.concatenate([W[0] - W[2], W[1], 2.0 * W[2]], axis=0)


def kernel(X, edge_index, edge_weight, Wxz, bxz, Whz, bhz, Wxr, bxr, Whr, bhr,
           Wxh, bxh, Whh, bhh):
    src = edge_index[0]
    dst = edge_index[1]
    deg = jax.ops.segment_sum(edge_weight, src, num_segments=N_NODES)
    dinv = jnp.where(deg > 0, deg ** -0.5, 0.0)
    w = -dinv[src] * edge_weight * dinv[dst]

    src3 = src.reshape(NW, NSEG, SEGCHUNK, CHUNK)
    dst3 = dst.reshape(NW, NSEG, SEGCHUNK, CHUNK)
    w3 = w.reshape(NW, NSEG, SEGCHUNK, CHUNK)

    def S(v):
        parts = _lap_sc(v, src3, dst3, w3)
        return parts[0] + parts[1]

    batches, seq_len, num_nodes, _ = X.shape

    Wx_cat = jnp.concatenate(
        [_cat_weights(Wxz), _cat_weights(Wxr), _cat_weights(Wxh)], axis=1)
    bx_cat = jnp.concatenate([bxz, bxr, bxh])
    Whzr_cat = jnp.concatenate([_cat_weights(Whz), _cat_weights(Whr)], axis=1)
    bh_zr = jnp.concatenate([bhz, bhr])
    Whh_cat = _cat_weights(Whh)

    Gx = []
    for t in range(seq_len):
        Xt = X[0, t]
        X1 = S(Xt)
        X2 = S(X1)
        basis = jnp.concatenate([Xt, X1, X2], axis=1)
        Gx.append(basis @ Wx_cat + bx_cat)

    H = jnp.zeros((num_nodes, HID), dtype=X.dtype)
    states = []
    for t in range(seq_len):
        H1 = S(H)
        H2 = S(H1)
        Bh = jnp.concatenate([H, H1, H2], axis=1)
        hmm = Bh @ Whzr_cat + bh_zr
        Z, HR = _gru_zr(Gx[t][:, :2 * HID], hmm, H)
        R1 = S(HR)
        R2 = S(R1)
        Bhr = jnp.concatenate([HR, R1, R2], axis=1)
        hrmm = Bhr @ Whh_cat + bhh
        H = _gru_h(Gx[t][:, 2 * HID:], hrmm, Z, H)
        states.append(H)

    out = jnp.stack(states, axis=0)[None]
    return out, H[None]
